# baseline (speedup 1.0000x reference)
import jax
import jax.numpy as jnp
from jax import lax
from jax.experimental import pallas as pl
from jax.experimental.pallas import tpu as pltpu, tpu_sc as plsc

NC, NS, L = 2, 16, 16
NB = 256
INT_MIN_I32 = -2147483648


def _fc_bn_body(x_ref, w_ref, b_ref, g_ref, be_ref, bn_ref, bnT_ref):
    N, D = bn_ref.shape
    H = N // 2

    def fold(c):
        return c[:, :D] + c[:, D:]

    def dup(c):
        return jnp.concatenate([c, c], axis=1)

    wt = w_ref[...].T
    top = jnp.dot(x_ref[0:H, :], wt, preferred_element_type=jnp.float32)
    bot = jnp.dot(x_ref[H:N, :], wt, preferred_element_type=jnp.float32)
    embp = jnp.concatenate([top, bot], axis=1) + dup(b_ref[...])

    mean = fold(jnp.sum(embp, axis=0, keepdims=True)) / N
    dev = embp - dup(mean)
    var = fold(jnp.sum(dev * dev, axis=0, keepdims=True)) / N
    scale = jax.lax.rsqrt(var + 1e-5) * g_ref[...]
    bnp = dev * dup(scale) + dup(be_ref[...])
    bn_ref[0:H, :] = bnp[:, 0:D]
    bn_ref[H:N, :] = bnp[:, D:]
    bnT_ref[...] = bnp.T


def _unpack_body(hT_ref, hash_ref):
    N, D = hash_ref.shape
    H = N // 2
    t = hT_ref[...].T
    hash_ref[0:H, :] = t[:, 0:D]
    hash_ref[H:N, :] = t[:, D:]


def _sc_select_body(bnT_hbm, outT_hbm, vals, keys, hist, outv):
    D2, H = bnT_hbm.shape
    D = D2 // 2
    N = 2 * H
    K = N // 2
    NV = N // L

    wid = lax.axis_index("s") * NC + lax.axis_index("c")
    lane = lax.iota(jnp.int32, L)
    ones_i = jnp.ones((L,), jnp.int32)

    def do_column(c01, _):
        j = 2 * wid + c01
        pltpu.sync_copy(bnT_hbm.at[j], vals.at[pl.ds(0, H)])
        pltpu.sync_copy(bnT_hbm.at[j + D], vals.at[pl.ds(H, H)])

        def xform(i, _):
            v = vals[pl.ds(i * L, L)]
            r = lax.bitcast_convert_type(v, jnp.int32)
            sk = r ^ ((r >> 31) & jnp.int32(0x7FFFFFFF))
            sk = sk + jnp.where(sk == -1, jnp.int32(1), jnp.int32(0))
            keys[pl.ds(i * L, L)] = sk
            return 0

        lax.fori_loop(0, NV, xform, 0, unroll=8)

        P = jnp.int32(0)
        Kp = jnp.int32(K)
        for lvl in range(4):
            sh = 24 - 8 * lvl

            def clear(i, _):
                hist[pl.ds(i * L, L)] = jnp.zeros((L,), jnp.int32)
                return 0

            lax.fori_loop(0, NB * L // L, clear, 0, unroll=8)

            def sweep(i, _):
                sk = keys[pl.ds(i * L, L)]
                ub = lax.shift_right_logical(
                    sk ^ jnp.int32(INT_MIN_I32), jnp.int32(sh))
                idx = lane * NB + (ub & jnp.int32(NB - 1))
                if lvl == 0:
                    plsc.addupdate_scatter(hist, [idx], ones_i)
                else:
                    m = lax.shift_right_logical(ub, jnp.int32(8)) == P
                    plsc.addupdate_scatter(hist, [idx], ones_i, mask=m)
                return 0

            lax.fori_loop(0, NV, sweep, 0, unroll=8)

            def chunk(c2, carry):
                running, b_star, cnt_gt = carry
                c = NB // L - 1 - c2

                def comb(i2, acc):
                    return acc + hist[pl.ds(i2 * NB + c * L, L)]

                acc = lax.fori_loop(
                    0, L, comb, jnp.zeros((L,), jnp.int32), unroll=4)
                total = jnp.sum(acc)
                cum = plsc.cumsum(acc)
                suffix = running + (total - cum)
                cond = (suffix < Kp) & (suffix + acc >= Kp)
                bvec = c * L + lane
                b_star = jnp.maximum(
                    b_star, jnp.max(jnp.where(cond, bvec, jnp.int32(-1))))
                cnt_gt = jnp.maximum(
                    cnt_gt, jnp.max(jnp.where(cond, suffix, jnp.int32(-1))))
                return running + total, b_star, cnt_gt

            running, b_star, cnt_gt = lax.fori_loop(
                0, NB // L, chunk,
                (jnp.int32(0), jnp.int32(-1), jnp.int32(-1)))
            P = (P << 8) | b_star
            Kp = Kp - cnt_gt

        T = P ^ jnp.int32(INT_MIN_I32)

        def wb(i, r):
            sk = keys[pl.ds(i * L, L)]
            gt = sk > T
            eqm = sk == T
            eqi = jnp.where(eqm, jnp.int32(1), jnp.int32(0))
            pos = plsc.cumsum(eqi)
            plus = gt | (eqm & (pos <= r))
            outv[pl.ds(i * L, L)] = jnp.where(
                plus, jnp.float32(1.0), jnp.float32(-1.0))
            return r - jnp.sum(eqi)

        lax.fori_loop(0, NV, wb, Kp, unroll=8)

        pltpu.sync_copy(outv.at[pl.ds(0, H)], outT_hbm.at[j])
        pltpu.sync_copy(outv.at[pl.ds(H, H)], outT_hbm.at[j + D])
        return 0

    lax.fori_loop(0, 2, do_column, 0)


@jax.jit
def kernel(x, W, b, gamma, beta):
    N, G = x.shape
    D = W.shape[0]
    H = N // 2

    bn, bnT = pl.pallas_call(
        _fc_bn_body,
        out_shape=(
            jax.ShapeDtypeStruct((N, D), jnp.float32),
            jax.ShapeDtypeStruct((2 * D, H), jnp.float32),
        ),
    )(x, W, b.reshape(1, D), gamma.reshape(1, D), beta.reshape(1, D))

    mesh = plsc.VectorSubcoreMesh(
        core_axis_name="c", subcore_axis_name="s",
        num_cores=NC, num_subcores=NS)
    hashT = pl.kernel(
        _sc_select_body,
        out_type=jax.ShapeDtypeStruct((2 * D, H), jnp.float32),
        mesh=mesh,
        compiler_params=pltpu.CompilerParams(needs_layout_passes=False),
        scratch_types=[
            pltpu.VMEM((N,), jnp.float32),
            pltpu.VMEM((N,), jnp.int32),
            pltpu.VMEM((NB * L,), jnp.int32),
            pltpu.VMEM((N,), jnp.float32),
        ],
    )(bnT)

    hsh = pl.pallas_call(
        _unpack_body,
        out_shape=jax.ShapeDtypeStruct((N, D), jnp.float32),
    )(hashT)

    return bn, hsh


# speedup vs baseline: 1.0491x; 1.0491x over previous
import jax
import jax.numpy as jnp
from jax import lax
from jax.experimental import pallas as pl
from jax.experimental.pallas import tpu as pltpu, tpu_sc as plsc

NC, NS, L = 2, 16, 16
NB = 256
INT_MIN_I32 = -2147483648


def _fc_bn_body(x_ref, w_ref, b_ref, g_ref, be_ref, bn_ref, bnT_ref):
    N, D = bn_ref.shape
    H = N // 2

    def fold(c):
        return c[:, :D] + c[:, D:]

    def dup(c):
        return jnp.concatenate([c, c], axis=1)

    wt = w_ref[...].T
    top = jnp.dot(x_ref[0:H, :], wt, preferred_element_type=jnp.float32)
    bot = jnp.dot(x_ref[H:N, :], wt, preferred_element_type=jnp.float32)
    embp = jnp.concatenate([top, bot], axis=1) + dup(b_ref[...])

    mean = fold(jnp.sum(embp, axis=0, keepdims=True)) / N
    dev = embp - dup(mean)
    var = fold(jnp.sum(dev * dev, axis=0, keepdims=True)) / N
    scale = jax.lax.rsqrt(var + 1e-5) * g_ref[...]
    bnp = dev * dup(scale) + dup(be_ref[...])
    bn_ref[0:H, :] = bnp[:, 0:D]
    bn_ref[H:N, :] = bnp[:, D:]
    bnT_ref[...] = bnp.T


def _unpack_body(hT_ref, hash_ref):
    N, D = hash_ref.shape
    H = N // 2
    t = hT_ref[...].T
    hash_ref[0:H, :] = t[:, 0:D]
    hash_ref[H:N, :] = t[:, D:]


def _sc_select_body(bnT_hbm, outT_hbm, vals, keys, hist, outv):
    D2, H = bnT_hbm.shape
    D = D2 // 2
    N = 2 * H
    K = N // 2
    NV = N // L

    wid = lax.axis_index("s") * NC + lax.axis_index("c")
    lane = lax.iota(jnp.int32, L)
    ones_i = jnp.ones((L,), jnp.int32)

    def do_column(c01, _):
        j = 2 * wid + c01
        pltpu.sync_copy(bnT_hbm.at[j], vals.at[pl.ds(0, H)])
        pltpu.sync_copy(bnT_hbm.at[j + D], vals.at[pl.ds(H, H)])

        P = jnp.int32(0)
        Kp = jnp.int32(K)
        for lvl in range(4):
            sh = 24 - 8 * lvl

            def clear(i, _):
                hist[pl.ds(i * L, L)] = jnp.zeros((L,), jnp.int32)
                return 0

            lax.fori_loop(0, (NB * 17 + L - 1) // L, clear, 0, unroll=8)

            def sweep(i, _):
                if lvl == 0:
                    v = vals[pl.ds(i * L, L)]
                    r = lax.bitcast_convert_type(v, jnp.int32)
                    sk = r ^ ((r >> 31) & jnp.int32(0x7FFFFFFF))
                    sk = sk + jnp.where(
                        sk == -1, jnp.int32(1), jnp.int32(0))
                    keys[pl.ds(i * L, L)] = sk
                else:
                    sk = keys[pl.ds(i * L, L)]
                ub = lax.shift_right_logical(
                    sk ^ jnp.int32(INT_MIN_I32), jnp.int32(sh))
                bkt = ub & jnp.int32(NB - 1)
                idx = (bkt << 4) + bkt + lane
                if lvl == 0:
                    plsc.addupdate_scatter(hist, [idx], ones_i)
                else:
                    m = lax.shift_right_logical(ub, jnp.int32(8)) == P
                    plsc.addupdate_scatter(hist, [idx], ones_i, mask=m)
                return 0

            lax.fori_loop(0, NV, sweep, 0, unroll=8)

            def chunk(c2, carry):
                running, b_star, cnt_gt = carry
                c = NB // L - 1 - c2
                base = ((c * L + lane) << 4) + (c * L + lane)

                acc = jnp.zeros((L,), jnp.int32)
                for p in range(L):
                    acc = acc + plsc.load_gather(
                        hist, [base + jnp.int32(p)])
                total = jnp.sum(acc)
                cum = plsc.cumsum(acc)
                suffix = running + (total - cum)
                cond = (suffix < Kp) & (suffix + acc >= Kp)
                bvec = c * L + lane
                b_star = jnp.maximum(
                    b_star, jnp.max(jnp.where(cond, bvec, jnp.int32(-1))))
                cnt_gt = jnp.maximum(
                    cnt_gt, jnp.max(jnp.where(cond, suffix, jnp.int32(-1))))
                return running + total, b_star, cnt_gt

            running, b_star, cnt_gt = lax.fori_loop(
                0, NB // L, chunk,
                (jnp.int32(0), jnp.int32(-1), jnp.int32(-1)))
            P = (P << 8) | b_star
            Kp = Kp - cnt_gt

        T = P ^ jnp.int32(INT_MIN_I32)

        def wb(i, r):
            sk = keys[pl.ds(i * L, L)]
            gt = sk > T
            eqm = sk == T
            eqi = jnp.where(eqm, jnp.int32(1), jnp.int32(0))
            pos = plsc.cumsum(eqi)
            plus = gt | (eqm & (pos <= r))
            outv[pl.ds(i * L, L)] = jnp.where(
                plus, jnp.float32(1.0), jnp.float32(-1.0))
            return r - plsc.all_reduce_population_count(eqm)

        lax.fori_loop(0, NV, wb, jnp.broadcast_to(Kp, (L,)), unroll=8)

        pltpu.sync_copy(outv.at[pl.ds(0, H)], outT_hbm.at[j])
        pltpu.sync_copy(outv.at[pl.ds(H, H)], outT_hbm.at[j + D])
        return 0

    lax.fori_loop(0, 2, do_column, 0)


@jax.jit
def kernel(x, W, b, gamma, beta):
    N, G = x.shape
    D = W.shape[0]
    H = N // 2

    bn, bnT = pl.pallas_call(
        _fc_bn_body,
        out_shape=(
            jax.ShapeDtypeStruct((N, D), jnp.float32),
            jax.ShapeDtypeStruct((2 * D, H), jnp.float32),
        ),
    )(x, W, b.reshape(1, D), gamma.reshape(1, D), beta.reshape(1, D))

    mesh = plsc.VectorSubcoreMesh(
        core_axis_name="c", subcore_axis_name="s",
        num_cores=NC, num_subcores=NS)
    hashT = pl.kernel(
        _sc_select_body,
        out_type=jax.ShapeDtypeStruct((2 * D, H), jnp.float32),
        mesh=mesh,
        compiler_params=pltpu.CompilerParams(needs_layout_passes=False),
        scratch_types=[
            pltpu.VMEM((N,), jnp.float32),
            pltpu.VMEM((N,), jnp.int32),
            pltpu.VMEM((NB * L,), jnp.int32),
            pltpu.VMEM((N,), jnp.float32),
        ],
    )(bnT)

    hsh = pl.pallas_call(
        _unpack_body,
        out_shape=jax.ShapeDtypeStruct((N, D), jnp.float32),
    )(hashT)

    return bn, hsh


# speedup vs baseline: 2.2586x; 2.1529x over previous
import jax
import jax.numpy as jnp
from jax import lax
from jax.experimental import pallas as pl
from jax.experimental.pallas import tpu as pltpu, tpu_sc as plsc

NC, NS, L = 2, 16, 16
NB = 256
INT_MIN_I32 = -2147483648


def _fc_bn_body(x_ref, w_ref, b_ref, g_ref, be_ref, bn_ref, bnT_ref):
    N, D = bn_ref.shape
    H = N // 2

    def fold(c):
        return c[:, :D] + c[:, D:]

    def dup(c):
        return jnp.concatenate([c, c], axis=1)

    wt = w_ref[...].T
    top = jnp.dot(x_ref[0:H, :], wt, preferred_element_type=jnp.float32)
    bot = jnp.dot(x_ref[H:N, :], wt, preferred_element_type=jnp.float32)
    embp = jnp.concatenate([top, bot], axis=1) + dup(b_ref[...])

    mean = fold(jnp.sum(embp, axis=0, keepdims=True)) / N
    dev = embp - dup(mean)
    var = fold(jnp.sum(dev * dev, axis=0, keepdims=True)) / N
    scale = jax.lax.rsqrt(var + 1e-5) * g_ref[...]
    bnp = dev * dup(scale) + dup(be_ref[...])
    bn_ref[0:H, :] = bnp[:, 0:D]
    bn_ref[H:N, :] = bnp[:, D:]
    bnT_ref[...] = bnp.T


def _unpack_body(hT_ref, hash_ref):
    N, D = hash_ref.shape
    H = N // 2
    t = hT_ref[...].T
    hash_ref[0:H, :] = t[:, 0:D]
    hash_ref[H:N, :] = t[:, D:]


def _sc_select_body(bnT_hbm, outT_hbm, vals, keys, hist, outv):
    D2, H = bnT_hbm.shape
    D = D2 // 2
    N = 2 * H
    K = N // 2
    NV = N // L

    wid = lax.axis_index("s") * NC + lax.axis_index("c")
    lane = lax.iota(jnp.int32, L)
    ones_i = jnp.ones((L,), jnp.int32)

    def do_column(c01, _):
        j = 2 * wid + c01
        pltpu.sync_copy(bnT_hbm.at[j], vals.at[pl.ds(0, H)])
        pltpu.sync_copy(bnT_hbm.at[j + D], vals.at[pl.ds(H, H)])

        P = jnp.int32(0)
        Kp = jnp.int32(K)
        for lvl in range(4):
            sh = 24 - 8 * lvl

            @plsc.parallel_loop(0, (NB * 17 + L - 1) // L, unroll=8)
            def _clear(i):
                hist[pl.ds(i * L, L)] = jnp.zeros((L,), jnp.int32)

            @plsc.parallel_loop(0, NV, unroll=8)
            def _sweep(i):
                if lvl == 0:
                    v = vals[pl.ds(i * L, L)]
                    r = lax.bitcast_convert_type(v, jnp.int32)
                    sk = r ^ ((r >> 31) & jnp.int32(0x7FFFFFFF))
                    sk = sk + jnp.where(
                        sk == -1, jnp.int32(1), jnp.int32(0))
                    keys[pl.ds(i * L, L)] = sk
                else:
                    sk = keys[pl.ds(i * L, L)]
                ub = lax.shift_right_logical(
                    sk ^ jnp.int32(INT_MIN_I32), jnp.int32(sh))
                bkt = ub & jnp.int32(NB - 1)
                idx = (bkt << 4) + bkt + lane
                if lvl == 0:
                    plsc.addupdate_scatter(hist, [idx], ones_i)
                else:
                    m = lax.shift_right_logical(ub, jnp.int32(8)) == P
                    plsc.addupdate_scatter(hist, [idx], ones_i, mask=m)

            def chunk(c2, carry):
                running, b_star, cnt_gt = carry
                c = NB // L - 1 - c2
                base = ((c * L + lane) << 4) + (c * L + lane)

                acc = jnp.zeros((L,), jnp.int32)
                for p in range(L):
                    acc = acc + plsc.load_gather(
                        hist, [base + jnp.int32(p)])
                total = jnp.sum(acc)
                cum = plsc.cumsum(acc)
                suffix = running + (total - cum)
                cond = (suffix < Kp) & (suffix + acc >= Kp)
                bvec = c * L + lane
                b_star = jnp.maximum(
                    b_star, jnp.max(jnp.where(cond, bvec, jnp.int32(-1))))
                cnt_gt = jnp.maximum(
                    cnt_gt, jnp.max(jnp.where(cond, suffix, jnp.int32(-1))))
                return running + total, b_star, cnt_gt

            running, b_star, cnt_gt = lax.fori_loop(
                0, NB // L, chunk,
                (jnp.int32(0), jnp.int32(-1), jnp.int32(-1)))
            P = (P << 8) | b_star
            Kp = Kp - cnt_gt

        T = P ^ jnp.int32(INT_MIN_I32)

        @plsc.parallel_loop(0, NV, unroll=8,
                            carry=jnp.broadcast_to(Kp, (L,)))
        def _wb(i, r):
            sk = keys[pl.ds(i * L, L)]
            gt = sk > T
            eqm = sk == T
            eqi = jnp.where(eqm, jnp.int32(1), jnp.int32(0))
            pos = plsc.cumsum(eqi)
            plus = gt | (eqm & (pos <= r))
            outv[pl.ds(i * L, L)] = jnp.where(
                plus, jnp.float32(1.0), jnp.float32(-1.0))
            return r - plsc.all_reduce_population_count(eqm)

        pltpu.sync_copy(outv.at[pl.ds(0, H)], outT_hbm.at[j])
        pltpu.sync_copy(outv.at[pl.ds(H, H)], outT_hbm.at[j + D])
        return 0

    lax.fori_loop(0, 2, do_column, 0)


@jax.jit
def kernel(x, W, b, gamma, beta):
    N, G = x.shape
    D = W.shape[0]
    H = N // 2

    bn, bnT = pl.pallas_call(
        _fc_bn_body,
        out_shape=(
            jax.ShapeDtypeStruct((N, D), jnp.float32),
            jax.ShapeDtypeStruct((2 * D, H), jnp.float32),
        ),
    )(x, W, b.reshape(1, D), gamma.reshape(1, D), beta.reshape(1, D))

    mesh = plsc.VectorSubcoreMesh(
        core_axis_name="c", subcore_axis_name="s",
        num_cores=NC, num_subcores=NS)
    hashT = pl.kernel(
        _sc_select_body,
        out_type=jax.ShapeDtypeStruct((2 * D, H), jnp.float32),
        mesh=mesh,
        compiler_params=pltpu.CompilerParams(needs_layout_passes=False),
        scratch_types=[
            pltpu.VMEM((N,), jnp.float32),
            pltpu.VMEM((N,), jnp.int32),
            pltpu.VMEM((NB * L,), jnp.int32),
            pltpu.VMEM((N,), jnp.float32),
        ],
    )(bnT)

    hsh = pl.pallas_call(
        _unpack_body,
        out_shape=jax.ShapeDtypeStruct((N, D), jnp.float32),
    )(hashT)

    return bn, hsh


# speedup vs baseline: 2.3510x; 1.0409x over previous
import jax
import jax.numpy as jnp
from jax import lax
from jax.experimental import pallas as pl
from jax.experimental.pallas import tpu as pltpu, tpu_sc as plsc

NC, NS, L = 2, 16, 16
NB = 256
INT_MIN_I32 = -2147483648


def _fc_bn_body(x_ref, w_ref, b_ref, g_ref, be_ref, bnp_ref, bnT_ref):
    H, D2 = bnp_ref.shape
    D = D2 // 2
    N = 2 * H

    def fold(c):
        return c[:, :D] + c[:, D:]

    def dup(c):
        return jnp.concatenate([c, c], axis=1)

    wt = w_ref[...].T
    top = jnp.dot(x_ref[0:H, :], wt, preferred_element_type=jnp.float32)
    bot = jnp.dot(x_ref[H:N, :], wt, preferred_element_type=jnp.float32)
    embp = jnp.concatenate([top, bot], axis=1) + dup(b_ref[...])

    mean = fold(jnp.sum(embp, axis=0, keepdims=True)) / N
    dev = embp - dup(mean)
    var = fold(jnp.sum(dev * dev, axis=0, keepdims=True)) / N
    scale = jax.lax.rsqrt(var + 1e-5) * g_ref[...]
    bnp = dev * dup(scale) + dup(be_ref[...])
    bnp_ref[...] = bnp
    bnT_ref[...] = bnp.T


def _unpack_lanes_body(p_ref, out_ref):
    N, D = out_ref.shape
    H = N // 2
    p = p_ref[...]
    out_ref[0:H, :] = p[:, 0:D]
    out_ref[H:N, :] = p[:, D:]


def _unpack_body(hT_ref, hash_ref):
    N, D = hash_ref.shape
    H = N // 2
    t = hT_ref[...].T
    hash_ref[0:H, :] = t[:, 0:D]
    hash_ref[H:N, :] = t[:, D:]


def _sc_select_body(bnT_hbm, outT_hbm, vals, keys, hist, outv):
    D2, H = bnT_hbm.shape
    D = D2 // 2
    N = 2 * H
    K = N // 2
    NV = N // L

    wid = lax.axis_index("s") * NC + lax.axis_index("c")
    lane = lax.iota(jnp.int32, L)
    ones_i = jnp.ones((L,), jnp.int32)

    def do_column(c01, _):
        j = 2 * wid + c01
        pltpu.sync_copy(bnT_hbm.at[j], vals.at[pl.ds(0, H)])
        pltpu.sync_copy(bnT_hbm.at[j + D], vals.at[pl.ds(H, H)])

        P = jnp.int32(0)
        Kp = jnp.int32(K)
        for lvl in range(4):
            sh = 24 - 8 * lvl

            @plsc.parallel_loop(0, (NB * 17 + L - 1) // L, unroll=8)
            def _clear(i):
                hist[pl.ds(i * L, L)] = jnp.zeros((L,), jnp.int32)

            @plsc.parallel_loop(0, NV, unroll=8)
            def _sweep(i):
                if lvl == 0:
                    v = vals[pl.ds(i * L, L)]
                    r = lax.bitcast_convert_type(v, jnp.int32)
                    sk = r ^ ((r >> 31) & jnp.int32(0x7FFFFFFF))
                    sk = sk + jnp.where(
                        sk == -1, jnp.int32(1), jnp.int32(0))
                    keys[pl.ds(i * L, L)] = sk
                else:
                    sk = keys[pl.ds(i * L, L)]
                ub = lax.shift_right_logical(
                    sk ^ jnp.int32(INT_MIN_I32), jnp.int32(sh))
                bkt = ub & jnp.int32(NB - 1)
                idx = (bkt << 4) + bkt + lane
                if lvl == 0:
                    plsc.addupdate_scatter(hist, [idx], ones_i)
                else:
                    m = lax.shift_right_logical(ub, jnp.int32(8)) == P
                    plsc.addupdate_scatter(hist, [idx], ones_i, mask=m)

            def chunk(c2, carry):
                running, b_star, cnt_gt = carry
                c = NB // L - 1 - c2
                base = ((c * L + lane) << 4) + (c * L + lane)

                acc = jnp.zeros((L,), jnp.int32)
                for p in range(L):
                    acc = acc + plsc.load_gather(
                        hist, [base + jnp.int32(p)])
                total = jnp.sum(acc)
                cum = plsc.cumsum(acc)
                suffix = running + (total - cum)
                cond = (suffix < Kp) & (suffix + acc >= Kp)
                bvec = c * L + lane
                b_star = jnp.maximum(
                    b_star, jnp.max(jnp.where(cond, bvec, jnp.int32(-1))))
                cnt_gt = jnp.maximum(
                    cnt_gt, jnp.max(jnp.where(cond, suffix, jnp.int32(-1))))
                return running + total, b_star, cnt_gt

            running, b_star, cnt_gt = lax.fori_loop(
                0, NB // L, chunk,
                (jnp.int32(0), jnp.int32(-1), jnp.int32(-1)))
            P = (P << 8) | b_star
            Kp = Kp - cnt_gt

        T = P ^ jnp.int32(INT_MIN_I32)

        @plsc.parallel_loop(0, NV, unroll=8,
                            carry=jnp.broadcast_to(Kp, (L,)))
        def _wb(i, r):
            sk = keys[pl.ds(i * L, L)]
            gt = sk > T
            eqm = sk == T
            eqi = jnp.where(eqm, jnp.int32(1), jnp.int32(0))
            pos = plsc.cumsum(eqi)
            plus = gt | (eqm & (pos <= r))
            outv[pl.ds(i * L, L)] = jnp.where(
                plus, jnp.float32(1.0), jnp.float32(-1.0))
            return r - plsc.all_reduce_population_count(eqm)

        pltpu.sync_copy(outv.at[pl.ds(0, H)], outT_hbm.at[j])
        pltpu.sync_copy(outv.at[pl.ds(H, H)], outT_hbm.at[j + D])
        return 0

    lax.fori_loop(0, 2, do_column, 0)


@jax.jit
def kernel(x, W, b, gamma, beta):
    N, G = x.shape
    D = W.shape[0]
    H = N // 2

    bnp, bnT = pl.pallas_call(
        _fc_bn_body,
        out_shape=(
            jax.ShapeDtypeStruct((H, 2 * D), jnp.float32),
            jax.ShapeDtypeStruct((2 * D, H), jnp.float32),
        ),
    )(x, W, b.reshape(1, D), gamma.reshape(1, D), beta.reshape(1, D))

    mesh = plsc.VectorSubcoreMesh(
        core_axis_name="c", subcore_axis_name="s",
        num_cores=NC, num_subcores=NS)
    hashT = pl.kernel(
        _sc_select_body,
        out_type=jax.ShapeDtypeStruct((2 * D, H), jnp.float32),
        mesh=mesh,
        compiler_params=pltpu.CompilerParams(needs_layout_passes=False),
        scratch_types=[
            pltpu.VMEM((N,), jnp.float32),
            pltpu.VMEM((N,), jnp.int32),
            pltpu.VMEM((NB * L,), jnp.int32),
            pltpu.VMEM((N,), jnp.float32),
        ],
    )(bnT)

    bn = pl.pallas_call(
        _unpack_lanes_body,
        out_shape=jax.ShapeDtypeStruct((N, D), jnp.float32),
    )(bnp)

    hsh = pl.pallas_call(
        _unpack_body,
        out_shape=jax.ShapeDtypeStruct((N, D), jnp.float32),
    )(hashT)

    return bn, hsh


# speedup vs baseline: 2.4245x; 1.0313x over previous
import jax
import jax.numpy as jnp
from jax import lax
from jax.experimental import pallas as pl
from jax.experimental.pallas import tpu as pltpu, tpu_sc as plsc

NC, NS, L = 2, 16, 16
NB = 256
INT_MIN_I32 = -2147483648


def _fc_bn_body(x_ref, w_ref, b_ref, g_ref, be_ref, bnT_ref):
    D2, H = bnT_ref.shape
    D = D2 // 2
    N = 2 * H

    def fold(c):
        return c[:, :D] + c[:, D:]

    def dup(c):
        return jnp.concatenate([c, c], axis=1)

    wt = w_ref[...].T
    top = jnp.dot(x_ref[0:H, :], wt, preferred_element_type=jnp.float32)
    bot = jnp.dot(x_ref[H:N, :], wt, preferred_element_type=jnp.float32)
    embp = jnp.concatenate([top, bot], axis=1) + dup(b_ref[...])

    s1 = fold(jnp.sum(embp, axis=0, keepdims=True))
    s2 = fold(jnp.sum(embp * embp, axis=0, keepdims=True))
    mean = s1 / N
    var = s2 / N - mean * mean
    scale = jax.lax.rsqrt(var + 1e-5) * g_ref[...]
    shift = be_ref[...] - mean * scale
    bnp = embp * dup(scale) + dup(shift)
    bnT_ref[...] = bnp.T


def _unpack_lanes_body(bnT_ref, out_ref):
    N, D = out_ref.shape
    H = N // 2
    t = bnT_ref[...].T
    out_ref[0:H, :] = t[:, 0:D]
    out_ref[H:N, :] = t[:, D:]


def _unpack_body(hT_ref, hash_ref):
    N, D = hash_ref.shape
    H = N // 2
    t = hT_ref[...].T
    hash_ref[0:H, :] = t[:, 0:D]
    hash_ref[H:N, :] = t[:, D:]


def _sc_select_body(bnT_hbm, outT_hbm, vals, keys, hist, outv):
    D2, H = bnT_hbm.shape
    D = D2 // 2
    N = 2 * H
    K = N // 2
    NV = N // L

    wid = lax.axis_index("s") * NC + lax.axis_index("c")
    lane = lax.iota(jnp.int32, L)
    ones_i = jnp.ones((L,), jnp.int32)

    def do_column(c01, _):
        j = 2 * wid + c01
        pltpu.sync_copy(bnT_hbm.at[j], vals.at[pl.ds(0, H)])
        pltpu.sync_copy(bnT_hbm.at[j + D], vals.at[pl.ds(H, H)])

        P = jnp.int32(0)
        Kp = jnp.int32(K)
        for lvl in range(4):
            sh = 24 - 8 * lvl

            @plsc.parallel_loop(0, (NB * 17 + L - 1) // L, unroll=8)
            def _clear(i):
                hist[pl.ds(i * L, L)] = jnp.zeros((L,), jnp.int32)

            @plsc.parallel_loop(0, NV, unroll=8)
            def _sweep(i):
                if lvl == 0:
                    v = vals[pl.ds(i * L, L)]
                    r = lax.bitcast_convert_type(v, jnp.int32)
                    sk = r ^ ((r >> 31) & jnp.int32(0x7FFFFFFF))
                    sk = sk + jnp.where(
                        sk == -1, jnp.int32(1), jnp.int32(0))
                    keys[pl.ds(i * L, L)] = sk
                else:
                    sk = keys[pl.ds(i * L, L)]
                ub = lax.shift_right_logical(
                    sk ^ jnp.int32(INT_MIN_I32), jnp.int32(sh))
                bkt = ub & jnp.int32(NB - 1)
                idx = (bkt << 4) + bkt + lane
                if lvl == 0:
                    plsc.addupdate_scatter(hist, [idx], ones_i)
                else:
                    m = lax.shift_right_logical(ub, jnp.int32(8)) == P
                    plsc.addupdate_scatter(hist, [idx], ones_i, mask=m)

            def chunk(c2, carry):
                running, b_star, cnt_gt = carry
                c = NB // L - 1 - c2
                base = ((c * L + lane) << 4) + (c * L + lane)

                acc = jnp.zeros((L,), jnp.int32)
                for p in range(L):
                    acc = acc + plsc.load_gather(
                        hist, [base + jnp.int32(p)])
                total = jnp.sum(acc)
                cum = plsc.cumsum(acc)
                suffix = running + (total - cum)
                cond = (suffix < Kp) & (suffix + acc >= Kp)
                bvec = c * L + lane
                b_star = jnp.maximum(
                    b_star, jnp.max(jnp.where(cond, bvec, jnp.int32(-1))))
                cnt_gt = jnp.maximum(
                    cnt_gt, jnp.max(jnp.where(cond, suffix, jnp.int32(-1))))
                return running + total, b_star, cnt_gt

            running, b_star, cnt_gt = lax.fori_loop(
                0, NB // L, chunk,
                (jnp.int32(0), jnp.int32(-1), jnp.int32(-1)))
            P = (P << 8) | b_star
            Kp = Kp - cnt_gt

        T = P ^ jnp.int32(INT_MIN_I32)

        @plsc.parallel_loop(0, NV, unroll=8,
                            carry=jnp.broadcast_to(Kp, (L,)))
        def _wb(i, r):
            sk = keys[pl.ds(i * L, L)]
            gt = sk > T
            eqm = sk == T
            eqi = jnp.where(eqm, jnp.int32(1), jnp.int32(0))
            pos = plsc.cumsum(eqi)
            plus = gt | (eqm & (pos <= r))
            outv[pl.ds(i * L, L)] = jnp.where(
                plus, jnp.float32(1.0), jnp.float32(-1.0))
            return r - plsc.all_reduce_population_count(eqm)

        pltpu.sync_copy(outv.at[pl.ds(0, H)], outT_hbm.at[j])
        pltpu.sync_copy(outv.at[pl.ds(H, H)], outT_hbm.at[j + D])
        return 0

    lax.fori_loop(0, 2, do_column, 0)


@jax.jit
def kernel(x, W, b, gamma, beta):
    N, G = x.shape
    D = W.shape[0]
    H = N // 2

    bnT = pl.pallas_call(
        _fc_bn_body,
        out_shape=jax.ShapeDtypeStruct((2 * D, H), jnp.float32),
    )(x, W, b.reshape(1, D), gamma.reshape(1, D), beta.reshape(1, D))

    mesh = plsc.VectorSubcoreMesh(
        core_axis_name="c", subcore_axis_name="s",
        num_cores=NC, num_subcores=NS)
    hashT = pl.kernel(
        _sc_select_body,
        out_type=jax.ShapeDtypeStruct((2 * D, H), jnp.float32),
        mesh=mesh,
        compiler_params=pltpu.CompilerParams(needs_layout_passes=False),
        scratch_types=[
            pltpu.VMEM((N,), jnp.float32),
            pltpu.VMEM((N,), jnp.int32),
            pltpu.VMEM((NB * L,), jnp.int32),
            pltpu.VMEM((N,), jnp.float32),
        ],
    )(bnT)

    bn = pl.pallas_call(
        _unpack_lanes_body,
        out_shape=jax.ShapeDtypeStruct((N, D), jnp.float32),
    )(bnT)

    hsh = pl.pallas_call(
        _unpack_body,
        out_shape=jax.ShapeDtypeStruct((N, D), jnp.float32),
    )(hashT)

    return bn, hsh


# speedup vs baseline: 2.5725x; 1.0610x over previous
import jax
import jax.numpy as jnp
from jax import lax
from jax.experimental import pallas as pl
from jax.experimental.pallas import tpu as pltpu, tpu_sc as plsc

NC, NS, L = 2, 16, 16
NB = 256
INT_MIN_I32 = -2147483648


def _fc_bn_body(x_ref, w_ref, b_ref, g_ref, be_ref, bnT_ref):
    D2, H = bnT_ref.shape
    D = D2 // 2
    N = 2 * H

    def fold(c):
        return c[:, :D] + c[:, D:]

    def dup(c):
        return jnp.concatenate([c, c], axis=1)

    wt = w_ref[...].T
    top = jnp.dot(x_ref[0:H, :], wt, preferred_element_type=jnp.float32)
    bot = jnp.dot(x_ref[H:N, :], wt, preferred_element_type=jnp.float32)
    embp = jnp.concatenate([top, bot], axis=1) + dup(b_ref[...])

    s1 = fold(jnp.sum(embp, axis=0, keepdims=True))
    s2 = fold(jnp.sum(embp * embp, axis=0, keepdims=True))
    mean = s1 / N
    var = s2 / N - mean * mean
    scale = jax.lax.rsqrt(var + 1e-5) * g_ref[...]
    shift = be_ref[...] - mean * scale
    bnp = embp * dup(scale) + dup(shift)
    bnT_ref[...] = bnp.T


def _unpack_lanes_body(bnT_ref, out_ref):
    N, D = out_ref.shape
    H = N // 2
    t = bnT_ref[...].T
    out_ref[0:H, :] = t[:, 0:D]
    out_ref[H:N, :] = t[:, D:]


def _unpack_body(hT_ref, hash_ref):
    N, D = hash_ref.shape
    H = N // 2
    t = hT_ref[...].T
    hash_ref[0:H, :] = t[:, 0:D]
    hash_ref[H:N, :] = t[:, D:]


def _sc_select_body(bnT_hbm, outT_hbm, vals, keys, hist, outv, cbuf, ibuf):
    D2, H = bnT_hbm.shape
    D = D2 // 2
    N = 2 * H
    K = N // 2
    NV = N // L

    wid = lax.axis_index("s") * NC + lax.axis_index("c")
    lane = lax.iota(jnp.int32, L)
    ones_i = jnp.ones((L,), jnp.int32)

    def clear_hist():
        @plsc.parallel_loop(0, (NB * 17 + L - 1) // L, unroll=8)
        def _clear(i):
            hist[pl.ds(i * L, L)] = jnp.zeros((L,), jnp.int32)

    def scan_hist(Kp):
        def chunk(c2, carry):
            running, b_star, cnt_gt = carry
            c = NB // L - 1 - c2
            base = ((c * L + lane) << 4) + (c * L + lane)

            acc = jnp.zeros((L,), jnp.int32)
            for p in range(L):
                acc = acc + plsc.load_gather(hist, [base + jnp.int32(p)])
            total = jnp.sum(acc)
            cum = plsc.cumsum(acc)
            suffix = running + (total - cum)
            cond = (suffix < Kp) & (suffix + acc >= Kp)
            bvec = c * L + lane
            b_star = jnp.maximum(
                b_star, jnp.max(jnp.where(cond, bvec, jnp.int32(-1))))
            cnt_gt = jnp.maximum(
                cnt_gt, jnp.max(jnp.where(cond, suffix, jnp.int32(-1))))
            return running + total, b_star, cnt_gt

        return lax.fori_loop(
            0, NB // L, chunk,
            (jnp.int32(0), jnp.int32(-1), jnp.int32(-1)))

    def do_column(c01, _):
        j = 2 * wid + c01
        pltpu.sync_copy(bnT_hbm.at[j], vals.at[pl.ds(0, H)])
        pltpu.sync_copy(bnT_hbm.at[j + D], vals.at[pl.ds(H, H)])


        clear_hist()

        @plsc.parallel_loop(0, NV, unroll=8)
        def _sweep0(i):
            v = vals[pl.ds(i * L, L)]
            r = lax.bitcast_convert_type(v, jnp.int32)
            sk = r ^ ((r >> 31) & jnp.int32(0x7FFFFFFF))
            sk = sk + jnp.where(sk == -1, jnp.int32(1), jnp.int32(0))
            keys[pl.ds(i * L, L)] = sk
            ub = lax.shift_right_logical(
                sk ^ jnp.int32(INT_MIN_I32), jnp.int32(24))
            bkt = ub & jnp.int32(NB - 1)
            plsc.addupdate_scatter(hist, [(bkt << 4) + bkt + lane], ones_i)

        _, b_star, cnt_gt = scan_hist(jnp.int32(K))
        P = b_star
        Kp = jnp.int32(K) - cnt_gt

        clear_hist()

        @plsc.parallel_loop(0, NV, unroll=8,
                            carry=jnp.zeros((L,), jnp.int32))
        def _sweep1(i, off):
            sk = keys[pl.ds(i * L, L)]
            ub = lax.shift_right_logical(
                sk ^ jnp.int32(INT_MIN_I32), jnp.int32(16))
            m = lax.shift_right_logical(ub, jnp.int32(8)) == P
            bkt = ub & jnp.int32(NB - 1)
            plsc.addupdate_scatter(
                hist, [(bkt << 4) + bkt + lane], ones_i, mask=m)
            mi = jnp.where(m, jnp.int32(1), jnp.int32(0))
            dest = off + plsc.cumsum(mi) - mi
            plsc.store_scatter(cbuf, [dest], sk, mask=m)
            plsc.store_scatter(ibuf, [dest], i * L + lane, mask=m)
            return off + plsc.all_reduce_population_count(m)

        M1 = jnp.max(_sweep1)
        M1V = (M1 + (L - 1)) >> 4

        _, b_star, cnt_gt = scan_hist(Kp)
        P = (P << 8) | b_star
        Kp = Kp - cnt_gt

        for lvl in (2, 3):
            sh = 24 - 8 * lvl
            clear_hist()

            @plsc.parallel_loop(0, M1V, unroll=4)
            def _sweepc(i):
                sk = cbuf[pl.ds(i * L, L)]
                valid = (i * L + lane) < M1
                ub = lax.shift_right_logical(
                    sk ^ jnp.int32(INT_MIN_I32), jnp.int32(sh))
                m = valid & (
                    lax.shift_right_logical(ub, jnp.int32(8)) == P)
                bkt = ub & jnp.int32(NB - 1)
                plsc.addupdate_scatter(
                    hist, [(bkt << 4) + bkt + lane], ones_i, mask=m)

            _, b_star, cnt_gt = scan_hist(Kp)
            P = (P << 8) | b_star
            Kp = Kp - cnt_gt

        T = P ^ jnp.int32(INT_MIN_I32)

        @plsc.parallel_loop(0, M1V, unroll=4,
                            carry=(jnp.broadcast_to(Kp, (L,)),
                                   jnp.full((L,), -1, jnp.int32)))
        def _ties(i, carry):
            r, smax = carry
            sk = cbuf[pl.ds(i * L, L)]
            iv = ibuf[pl.ds(i * L, L)]
            eqm = ((i * L + lane) < M1) & (sk == T)
            eqi = jnp.where(eqm, jnp.int32(1), jnp.int32(0))
            pos = plsc.cumsum(eqi)
            sel = eqm & (pos <= r)
            smax = jnp.maximum(smax, jnp.where(sel, iv, jnp.int32(-1)))
            return r - plsc.all_reduce_population_count(eqm), smax

        S = jnp.max(_ties[1])

        @plsc.parallel_loop(0, NV, unroll=8)
        def _wb(i):
            sk = keys[pl.ds(i * L, L)]
            plus = (sk > T) | ((sk == T) & ((i * L + lane) <= S))
            outv[pl.ds(i * L, L)] = jnp.where(
                plus, jnp.float32(1.0), jnp.float32(-1.0))

        pltpu.sync_copy(outv.at[pl.ds(0, H)], outT_hbm.at[j])
        pltpu.sync_copy(outv.at[pl.ds(H, H)], outT_hbm.at[j + D])
        return 0

    lax.fori_loop(0, 2, do_column, 0)


@jax.jit
def kernel(x, W, b, gamma, beta):
    N, G = x.shape
    D = W.shape[0]
    H = N // 2

    bnT = pl.pallas_call(
        _fc_bn_body,
        out_shape=jax.ShapeDtypeStruct((2 * D, H), jnp.float32),
    )(x, W, b.reshape(1, D), gamma.reshape(1, D), beta.reshape(1, D))

    mesh = plsc.VectorSubcoreMesh(
        core_axis_name="c", subcore_axis_name="s",
        num_cores=NC, num_subcores=NS)
    hashT = pl.kernel(
        _sc_select_body,
        out_type=jax.ShapeDtypeStruct((2 * D, H), jnp.float32),
        mesh=mesh,
        compiler_params=pltpu.CompilerParams(needs_layout_passes=False),
        scratch_types=[
            pltpu.VMEM((N,), jnp.float32),
            pltpu.VMEM((N,), jnp.int32),
            pltpu.VMEM((NB * 17 + L,), jnp.int32),
            pltpu.VMEM((N,), jnp.float32),
            pltpu.VMEM((N,), jnp.int32),
            pltpu.VMEM((N,), jnp.int32),
        ],
    )(bnT)

    bn = pl.pallas_call(
        _unpack_lanes_body,
        out_shape=jax.ShapeDtypeStruct((N, D), jnp.float32),
    )(bnT)

    hsh = pl.pallas_call(
        _unpack_body,
        out_shape=jax.ShapeDtypeStruct((N, D), jnp.float32),
    )(hashT)

    return bn, hsh


# speedup vs baseline: 2.6258x; 1.0207x over previous
import jax
import jax.numpy as jnp
from jax import lax
from jax.experimental import pallas as pl
from jax.experimental.pallas import tpu as pltpu, tpu_sc as plsc

NC, NS, L = 2, 16, 16
NB = 256
INT_MIN_I32 = -2147483648


def _fc_bn_body(x_ref, w_ref, b_ref, g_ref, be_ref, bnT_ref):
    D2, H = bnT_ref.shape
    D = D2 // 2
    N = 2 * H

    def fold(c):
        return c[:, :D] + c[:, D:]

    def dup(c):
        return jnp.concatenate([c, c], axis=1)

    wt = w_ref[...].T
    top = jnp.dot(x_ref[0:H, :], wt, preferred_element_type=jnp.float32)
    bot = jnp.dot(x_ref[H:N, :], wt, preferred_element_type=jnp.float32)
    embp = jnp.concatenate([top, bot], axis=1) + dup(b_ref[...])

    s1 = fold(jnp.sum(embp, axis=0, keepdims=True))
    s2 = fold(jnp.sum(embp * embp, axis=0, keepdims=True))
    mean = s1 / N
    var = s2 / N - mean * mean
    scale = jax.lax.rsqrt(var + 1e-5) * g_ref[...]
    shift = be_ref[...] - mean * scale
    bnp = embp * dup(scale) + dup(shift)
    bnT_ref[...] = bnp.T


def _unpack_lanes_body(bnT_ref, out_ref):
    N, D = out_ref.shape
    H = N // 2
    t = bnT_ref[...].T
    out_ref[0:H, :] = t[:, 0:D]
    out_ref[H:N, :] = t[:, D:]


def _unpack_body(hT_ref, hash_ref):
    N, D = hash_ref.shape
    H = N // 2
    t = hT_ref[...].T
    hash_ref[0:H, :] = t[:, 0:D]
    hash_ref[H:N, :] = t[:, D:]


def _sc_select_body(bnT_hbm, outT_hbm, vals, keys, hist, outv, cbuf, ibuf,
                    sem_in, sem_out):
    D2, H = bnT_hbm.shape
    D = D2 // 2
    N = 2 * H
    K = N // 2
    NV = N // L

    wid = lax.axis_index("s") * NC + lax.axis_index("c")
    lane = lax.iota(jnp.int32, L)
    ones_i = jnp.ones((L,), jnp.int32)

    def clear_hist():
        @plsc.parallel_loop(0, (NB * 17 + L - 1) // L, unroll=8)
        def _clear(i):
            hist[pl.ds(i * L, L)] = jnp.zeros((L,), jnp.int32)

    def scan_hist(Kp):
        def chunk(c2, carry):
            running, b_star, cnt_gt = carry
            c = NB // L - 1 - c2
            base = ((c * L + lane) << 4) + (c * L + lane)

            acc = jnp.zeros((L,), jnp.int32)
            for p in range(L):
                acc = acc + plsc.load_gather(hist, [base + jnp.int32(p)])
            total = jnp.sum(acc)
            cum = plsc.cumsum(acc)
            suffix = running + (total - cum)
            cond = (suffix < Kp) & (suffix + acc >= Kp)
            bvec = c * L + lane
            b_star = jnp.maximum(
                b_star, jnp.max(jnp.where(cond, bvec, jnp.int32(-1))))
            cnt_gt = jnp.maximum(
                cnt_gt, jnp.max(jnp.where(cond, suffix, jnp.int32(-1))))
            return running + total, b_star, cnt_gt

        return lax.fori_loop(
            0, NB // L, chunk,
            (jnp.int32(0), jnp.int32(-1), jnp.int32(-1)))

    def do_column(j, base):

        clear_hist()

        @plsc.parallel_loop(0, NV, unroll=8)
        def _sweep0(i):
            v = vals[pl.ds(base + i * L, L)]
            r = lax.bitcast_convert_type(v, jnp.int32)
            sk = r ^ ((r >> 31) & jnp.int32(0x7FFFFFFF))
            sk = sk + jnp.where(sk == -1, jnp.int32(1), jnp.int32(0))
            keys[pl.ds(i * L, L)] = sk
            ub = lax.shift_right_logical(
                sk ^ jnp.int32(INT_MIN_I32), jnp.int32(24))
            bkt = ub & jnp.int32(NB - 1)
            plsc.addupdate_scatter(hist, [(bkt << 4) + bkt + lane], ones_i)

        _, b_star, cnt_gt = scan_hist(jnp.int32(K))
        P = b_star
        Kp = jnp.int32(K) - cnt_gt

        clear_hist()

        @plsc.parallel_loop(0, NV, unroll=8,
                            carry=jnp.zeros((L,), jnp.int32))
        def _sweep1(i, off):
            sk = keys[pl.ds(i * L, L)]
            ub = lax.shift_right_logical(
                sk ^ jnp.int32(INT_MIN_I32), jnp.int32(16))
            m = lax.shift_right_logical(ub, jnp.int32(8)) == P
            bkt = ub & jnp.int32(NB - 1)
            plsc.addupdate_scatter(
                hist, [(bkt << 4) + bkt + lane], ones_i, mask=m)
            mi = jnp.where(m, jnp.int32(1), jnp.int32(0))
            dest = off + plsc.cumsum(mi) - mi
            plsc.store_scatter(cbuf, [dest], sk, mask=m)
            plsc.store_scatter(ibuf, [dest], i * L + lane, mask=m)
            return off + plsc.all_reduce_population_count(m)

        M1 = jnp.max(_sweep1)
        M1V = (M1 + (L - 1)) >> 4

        _, b_star, cnt_gt = scan_hist(Kp)
        P = (P << 8) | b_star
        Kp = Kp - cnt_gt

        for lvl in (2, 3):
            sh = 24 - 8 * lvl
            clear_hist()

            @plsc.parallel_loop(0, M1V, unroll=4)
            def _sweepc(i):
                sk = cbuf[pl.ds(i * L, L)]
                valid = (i * L + lane) < M1
                ub = lax.shift_right_logical(
                    sk ^ jnp.int32(INT_MIN_I32), jnp.int32(sh))
                m = valid & (
                    lax.shift_right_logical(ub, jnp.int32(8)) == P)
                bkt = ub & jnp.int32(NB - 1)
                plsc.addupdate_scatter(
                    hist, [(bkt << 4) + bkt + lane], ones_i, mask=m)

            _, b_star, cnt_gt = scan_hist(Kp)
            P = (P << 8) | b_star
            Kp = Kp - cnt_gt

        T = P ^ jnp.int32(INT_MIN_I32)

        @plsc.parallel_loop(0, M1V, unroll=4,
                            carry=(jnp.broadcast_to(Kp, (L,)),
                                   jnp.full((L,), -1, jnp.int32)))
        def _ties(i, carry):
            r, smax = carry
            sk = cbuf[pl.ds(i * L, L)]
            iv = ibuf[pl.ds(i * L, L)]
            eqm = ((i * L + lane) < M1) & (sk == T)
            eqi = jnp.where(eqm, jnp.int32(1), jnp.int32(0))
            pos = plsc.cumsum(eqi)
            sel = eqm & (pos <= r)
            smax = jnp.maximum(smax, jnp.where(sel, iv, jnp.int32(-1)))
            return r - plsc.all_reduce_population_count(eqm), smax

        S = jnp.max(_ties[1])

        @plsc.parallel_loop(0, NV, unroll=8)
        def _wb(i):
            sk = keys[pl.ds(i * L, L)]
            plus = (sk > T) | ((sk == T) & ((i * L + lane) <= S))
            outv[pl.ds(base + i * L, L)] = jnp.where(
                plus, jnp.float32(1.0), jnp.float32(-1.0))

    j0 = 2 * wid
    j1 = j0 + 1
    pltpu.sync_copy(bnT_hbm.at[j0], vals.at[pl.ds(0, H)])
    pltpu.sync_copy(bnT_hbm.at[j0 + D], vals.at[pl.ds(H, H)])
    i1a = pltpu.async_copy(bnT_hbm.at[j1], vals.at[pl.ds(N, H)], sem_in)
    i1b = pltpu.async_copy(
        bnT_hbm.at[j1 + D], vals.at[pl.ds(N + H, H)], sem_in)
    do_column(j0, 0)
    o0a = pltpu.async_copy(outv.at[pl.ds(0, H)], outT_hbm.at[j0], sem_out)
    o0b = pltpu.async_copy(
        outv.at[pl.ds(H, H)], outT_hbm.at[j0 + D], sem_out)
    i1a.wait()
    i1b.wait()
    do_column(j1, N)
    o1a = pltpu.async_copy(outv.at[pl.ds(N, H)], outT_hbm.at[j1], sem_out)
    o1b = pltpu.async_copy(
        outv.at[pl.ds(N + H, H)], outT_hbm.at[j1 + D], sem_out)
    o0a.wait()
    o0b.wait()
    o1a.wait()
    o1b.wait()


@jax.jit
def kernel(x, W, b, gamma, beta):
    N, G = x.shape
    D = W.shape[0]
    H = N // 2

    bnT = pl.pallas_call(
        _fc_bn_body,
        out_shape=jax.ShapeDtypeStruct((2 * D, H), jnp.float32),
    )(x, W, b.reshape(1, D), gamma.reshape(1, D), beta.reshape(1, D))

    mesh = plsc.VectorSubcoreMesh(
        core_axis_name="c", subcore_axis_name="s",
        num_cores=NC, num_subcores=NS)
    hashT = pl.kernel(
        _sc_select_body,
        out_type=jax.ShapeDtypeStruct((2 * D, H), jnp.float32),
        mesh=mesh,
        compiler_params=pltpu.CompilerParams(needs_layout_passes=False),
        scratch_types=[
            pltpu.VMEM((2 * N,), jnp.float32),
            pltpu.VMEM((N,), jnp.int32),
            pltpu.VMEM((NB * 17 + L,), jnp.int32),
            pltpu.VMEM((2 * N,), jnp.float32),
            pltpu.VMEM((N,), jnp.int32),
            pltpu.VMEM((N,), jnp.int32),
            pltpu.SemaphoreType.DMA,
            pltpu.SemaphoreType.DMA,
        ],
    )(bnT)

    bn = pl.pallas_call(
        _unpack_lanes_body,
        out_shape=jax.ShapeDtypeStruct((N, D), jnp.float32),
    )(bnT)

    hsh = pl.pallas_call(
        _unpack_body,
        out_shape=jax.ShapeDtypeStruct((N, D), jnp.float32),
    )(hashT)

    return bn, hsh
